# baseline (speedup 1.0000x reference)
import functools

import jax
import jax.numpy as jnp
from jax import lax
from jax.experimental import pallas as pl
from jax.experimental.pallas import tpu as pltpu
from jax.experimental.pallas import tpu_sc as plsc

NC, NS, L = 2, 16, 16
NW = NC * NS
C = 128

_SC_PARAMS = pltpu.CompilerParams(needs_layout_passes=False,
                                  use_tc_tiling_on_sc=False)
_MESH = plsc.VectorSubcoreMesh(core_axis_name="c", subcore_axis_name="s")


def _wid():
    return lax.axis_index("c") * NS + lax.axis_index("s")


def _load_edge_slice(ei_hbm, row, wid, EWr, EW, buf_v, padval):
    pltpu.sync_copy(ei_hbm.at[row, pl.ds(wid * EWr, EWr)],
                    buf_v.at[pl.ds(0, EWr)])
    padv = jnp.full((L,), padval, jnp.int32)
    for i in range(EWr, EW, L):
        buf_v[pl.ds(i, L)] = padv


def _make_deg_kernel(NP, EWr, EW, PAD):
    SL = NP // NS

    @functools.partial(
        pl.kernel, mesh=_MESH,
        out_type=jax.ShapeDtypeStruct((NC, NP), jnp.float32),
        scratch_types=[
            pltpu.VMEM((EW,), jnp.int32),
            pltpu.VMEM((NP,), jnp.float32),
            pltpu.VMEM((SL,), jnp.float32),
            pltpu.VMEM((SL,), jnp.float32),
            pltpu.MemorySpace.VMEM_SHARED((NS, NP), jnp.float32),
        ],
        compiler_params=_SC_PARAMS,
    )
    def deg_kernel(ei_hbm, deg_hbm, didx_v, cnt_v, acc_v, tmp_v, cnts_sh):
        c = lax.axis_index("c")
        s = lax.axis_index("s")
        wid = _wid()
        zero16 = jnp.zeros((L,), jnp.float32)
        ones16 = jnp.full((L,), 1.0, jnp.float32)

        def zbody(i, _):
            cnt_v[pl.ds(i * L, L)] = zero16
            return 0
        lax.fori_loop(0, NP // L, zbody, 0)

        _load_edge_slice(ei_hbm, 1, wid, EWr, EW, didx_v, PAD)

        def body(i, _):
            idx = didx_v[pl.ds(i * L, L)]
            plsc.addupdate_scatter(cnt_v, [idx], ones16)
            return 0
        lax.fori_loop(0, EW // L, body, 0)

        pltpu.sync_copy(cnt_v, cnts_sh.at[s])
        plsc.subcore_barrier()

        pltpu.sync_copy(cnts_sh.at[0, pl.ds(s * SL, SL)], acc_v)
        for t in range(1, NS):
            pltpu.sync_copy(cnts_sh.at[t, pl.ds(s * SL, SL)], tmp_v)

            def abody(i, _):
                sl = pl.ds(i * L, L)
                acc_v[sl] = acc_v[sl] + tmp_v[sl]
                return 0
            lax.fori_loop(0, SL // L, abody, 0)
        pltpu.sync_copy(acc_v, deg_hbm.at[c, pl.ds(s * SL, SL)])

    return deg_kernel


def _make_agg_kernel(NP, EWr, EW, H, PAD):
    RPT = NP // NS
    G = EW // C
    assert G % 2 == 0

    @functools.partial(
        pl.kernel, mesh=_MESH,
        out_type=jax.ShapeDtypeStruct((NC, NP, H), jnp.float32),
        scratch_types=[
            pltpu.VMEM((EW,), jnp.int32),
            pltpu.VMEM((EW,), jnp.int32),
            pltpu.VMEM((C,), jnp.int32),
            pltpu.VMEM((C, H), jnp.float32),
            pltpu.VMEM((C, H), jnp.float32),
            pltpu.VMEM((L, H), jnp.float32),
            pltpu.MemorySpace.VMEM_SHARED((NP, H), jnp.float32),
            pltpu.MemorySpace.VMEM_SHARED((NP, H), jnp.float32),
            pltpu.SemaphoreType.DMA,
            pltpu.SemaphoreType.DMA,
        ],
        compiler_params=_SC_PARAMS,
    )
    def agg_kernel(hs_hbm, ei_hbm, out_hbm,
                   sidx_v, didx_v, dc_v, rows0_v, rows1_v, zb_v,
                   hs_sh, acc_sh, sem0, sem1):
        c = lax.axis_index("c")
        s = lax.axis_index("s")
        wid = _wid()
        zero16 = jnp.zeros((L,), jnp.float32)
        for i in range(L):
            for j in range(H // L):
                zb_v[i, pl.ds(j * L, L)] = zero16

        _load_edge_slice(ei_hbm, 0, wid, EWr, EW, sidx_v, PAD)
        _load_edge_slice(ei_hbm, 1, wid, EWr, EW, didx_v, PAD)
        pltpu.sync_copy(hs_hbm.at[pl.ds(s * RPT, RPT)],
                        hs_sh.at[pl.ds(s * RPT, RPT)])

        def zc(i, _):
            pltpu.sync_copy(zb_v, acc_sh.at[pl.ds(s * RPT + i * L, L)])
            return 0
        lax.fori_loop(0, RPT // L, zc, 0)
        plsc.subcore_barrier()

        def scatter(g, rows_v):
            for j in range(C // L):
                dc_v[pl.ds(j * L, L)] = didx_v[pl.ds(g * C + j * L, L)]
            pltpu.sync_copy(rows_v, acc_sh.at[dc_v], add=True)

        pltpu.async_copy(hs_sh.at[sidx_v.at[pl.ds(0, C)]], rows0_v, sem0)

        def body(i, _):
            g0 = 2 * i
            g1 = g0 + 1
            g2 = g0 + 2
            pltpu.async_copy(hs_sh.at[sidx_v.at[pl.ds(g1 * C, C)]],
                             rows1_v, sem1)
            pltpu.make_async_copy(hs_sh.at[sidx_v.at[pl.ds(g0 * C, C)]],
                                  rows0_v, sem0).wait()
            scatter(g0, rows0_v)

            @pl.when(g2 < G)
            def _():
                pltpu.async_copy(hs_sh.at[sidx_v.at[pl.ds(g2 * C, C)]],
                                 rows0_v, sem0)
            pltpu.make_async_copy(hs_sh.at[sidx_v.at[pl.ds(g1 * C, C)]],
                                  rows1_v, sem1).wait()
            scatter(g1, rows1_v)
            return 0
        lax.fori_loop(0, G // 2, body, 0)
        plsc.subcore_barrier()

        pltpu.sync_copy(acc_sh.at[pl.ds(s * RPT, RPT)],
                        out_hbm.at[c, pl.ds(s * RPT, RPT)])

    return agg_kernel


def _make_scores_kernel(NP, EWr, EW, H, PAD):
    G = EW // C
    assert G % 2 == 0
    NSUB = C // L

    @functools.partial(
        pl.kernel, mesh=_MESH,
        out_type=jax.ShapeDtypeStruct((NW * EW,), jnp.float32),
        scratch_types=[
            pltpu.VMEM((EW,), jnp.int32),
            pltpu.VMEM((EW,), jnp.int32),
            pltpu.VMEM((C, H), jnp.float32),
            pltpu.VMEM((C, H), jnp.float32),
            pltpu.VMEM((C, H), jnp.float32),
            pltpu.VMEM((C, H), jnp.float32),
            pltpu.VMEM((C,), jnp.float32),
            pltpu.MemorySpace.VMEM_SHARED((NP, H), jnp.float32),
            pltpu.SemaphoreType.DMA,
            pltpu.SemaphoreType.DMA,
        ],
        compiler_params=_SC_PARAMS,
    )
    def scores_kernel(nr_hbm, ei_hbm, out_hbm,
                      aidx_v, bidx_v, ab0_v, bb0_v, ab1_v, bb1_v, sc_v,
                      nr_sh, sem0, sem1):
        s = lax.axis_index("s")
        wid = _wid()
        base = wid * EW
        lanes = lax.iota(jnp.int32, L)
        RPT = NP // NS

        def start(g, abuf, bbuf, sem):
            pltpu.async_copy(nr_sh.at[aidx_v.at[pl.ds(g * C, C)]], abuf, sem)
            pltpu.async_copy(nr_sh.at[bidx_v.at[pl.ds(g * C, C)]], bbuf, sem)

        def wait(g, abuf, bbuf, sem):
            pltpu.make_async_copy(nr_sh.at[aidx_v.at[pl.ds(g * C, C)]],
                                  abuf, sem).wait()
            pltpu.make_async_copy(nr_sh.at[bidx_v.at[pl.ds(g * C, C)]],
                                  bbuf, sem).wait()

        def compute(g, abuf, bbuf):
            def sub_body(sub, _):
                acc = jnp.zeros((L,), jnp.float32)
                for j in range(L):
                    e = sub * L + j
                    p = jnp.zeros((L,), jnp.float32)
                    for q in range(H // L):
                        sl = pl.ds(q * L, L)
                        p = p + abuf[e, sl] * bbuf[e, sl]
                    t = jnp.sum(p)
                    acc = jnp.where(lanes == j, t, acc)
                sc_v[pl.ds(sub * L, L)] = acc
                return 0
            lax.fori_loop(0, NSUB, sub_body, 0)
            pltpu.sync_copy(sc_v, out_hbm.at[pl.ds(base + g * C, C)])

        _load_edge_slice(ei_hbm, 0, wid, EWr, EW, aidx_v, PAD)
        _load_edge_slice(ei_hbm, 1, wid, EWr, EW, bidx_v, PAD)
        pltpu.sync_copy(nr_hbm.at[pl.ds(s * RPT, RPT)],
                        nr_sh.at[pl.ds(s * RPT, RPT)])
        plsc.subcore_barrier()
        start(0, ab0_v, bb0_v, sem0)

        def body(i, _):
            g0 = 2 * i
            g1 = g0 + 1
            g2 = g0 + 2
            start(g1, ab1_v, bb1_v, sem1)
            wait(g0, ab0_v, bb0_v, sem0)
            compute(g0, ab0_v, bb0_v)

            @pl.when(g2 < G)
            def _():
                start(g2, ab0_v, bb0_v, sem0)
            wait(g1, ab1_v, bb1_v, sem1)
            compute(g1, ab1_v, bb1_v)
            return 0
        lax.fori_loop(0, G // 2, body, 0)

    return scores_kernel


def _mm1_body(xp_ref, w1_ref, h1_ref):
    h1_ref[...] = jnp.dot(xp_ref[...], w1_ref[...],
                          preferred_element_type=jnp.float32)


def _dense1_body(deg2_ref, h1_ref, hs1_ref):
    deg = deg2_ref[0] + deg2_ref[1] + 1.0
    dinv = lax.rsqrt(deg)
    hs1_ref[...] = h1_ref[...] * dinv


def _dense2_body(deg2_ref, acc_ref, hs1_ref, w2_ref, b1_ref, hs2_ref):
    dinv = lax.rsqrt(deg2_ref[0] + deg2_ref[1] + 1.0)
    tot = acc_ref[0] + acc_ref[1] + hs1_ref[...]
    out1 = jnp.maximum(dinv * tot + b1_ref[...], 0.0)
    h2 = jnp.dot(out1, w2_ref[...], preferred_element_type=jnp.float32)
    hs2_ref[...] = h2 * dinv


def _dense3_body(deg2_ref, acc_ref, hs2_ref, b2_ref, wd1_ref, bd1_ref,
                 wd2_ref, bd2_ref, nr_ref, recon_ref):
    dinv = lax.rsqrt(deg2_ref[0] + deg2_ref[1] + 1.0)
    tot = acc_ref[0] + acc_ref[1] + hs2_ref[...]
    nr = jnp.maximum(dinv * tot + b2_ref[...], 0.0)
    nr_ref[...] = nr
    r = jnp.maximum(
        jnp.dot(nr, wd1_ref[...], preferred_element_type=jnp.float32)
        + bd1_ref[...], 0.0)
    recon_ref[...] = (
        jnp.dot(r, wd2_ref[...], preferred_element_type=jnp.float32)
        + bd2_ref[...])


def kernel(x, edge_index, batch, W1, b1, W2, b2, Wd1, bd1, Wd2, bd2):
    N, F = x.shape
    H = W1.shape[1]
    E = edge_index.shape[1]

    NP = ((N + NW * L - 1) // (NW * L)) * (NW * L)
    assert E % NW == 0
    EWr = E // NW
    EW = ((EWr + 2 * C - 1) // (2 * C)) * 2 * C

    ei = edge_index.astype(jnp.int32)
    xp = jnp.pad(x, ((0, NP - N), (0, 0)))

    deg_k = _make_deg_kernel(NP, EWr, EW, N)
    agg_k = _make_agg_kernel(NP, EWr, EW, H, N)
    scores_k = _make_scores_kernel(NP, EWr, EW, H, N)

    deg2 = deg_k(ei)
    deg2c = deg2.reshape(NC, NP, 1)

    h1 = pl.pallas_call(
        _mm1_body,
        out_shape=jax.ShapeDtypeStruct((NP, H), jnp.float32))(xp, W1)
    hs1 = pl.pallas_call(
        _dense1_body,
        out_shape=jax.ShapeDtypeStruct((NP, H), jnp.float32))(deg2c, h1)

    acc1 = agg_k(hs1, ei)

    hs2 = pl.pallas_call(
        _dense2_body,
        out_shape=jax.ShapeDtypeStruct((NP, H), jnp.float32))(
        deg2c, acc1, hs1, W2, b1.reshape(1, H))

    acc2 = agg_k(hs2, ei)

    nr, recon = pl.pallas_call(
        _dense3_body,
        out_shape=(jax.ShapeDtypeStruct((NP, H), jnp.float32),
                   jax.ShapeDtypeStruct((NP, F), jnp.float32)))(
        deg2c, acc2, hs2, b2.reshape(1, H),
        Wd1, bd1.reshape(1, H), Wd2, bd2.reshape(1, F))

    scores = scores_k(nr, ei)
    scores = scores.reshape(NW, EW)[:, :EWr].reshape(E)

    return (recon[:N], scores, nr[:N])


# speedup vs baseline: 1.0029x; 1.0029x over previous
import functools

import jax
import jax.numpy as jnp
from jax import lax
from jax.experimental import pallas as pl
from jax.experimental.pallas import tpu as pltpu
from jax.experimental.pallas import tpu_sc as plsc

NC, NS, L = 2, 16, 16
NW = NC * NS
C = 128

_SC_PARAMS = pltpu.CompilerParams(needs_layout_passes=False,
                                  use_tc_tiling_on_sc=False)
_MESH = plsc.VectorSubcoreMesh(core_axis_name="c", subcore_axis_name="s")


def _wid():
    return lax.axis_index("c") * NS + lax.axis_index("s")


def _load_edge_slice(ei_hbm, row, wid, EWr, EW, buf_v, padval):
    pltpu.sync_copy(ei_hbm.at[row, pl.ds(wid * EWr, EWr)],
                    buf_v.at[pl.ds(0, EWr)])
    padv = jnp.full((L,), padval, jnp.int32)
    for i in range(EWr, EW, L):
        buf_v[pl.ds(i, L)] = padv


def _make_deg_kernel(NP, EWr, EW, PAD):
    SL = NP // NS

    @functools.partial(
        pl.kernel, mesh=_MESH,
        out_type=jax.ShapeDtypeStruct((NC, NP), jnp.float32),
        scratch_types=[
            pltpu.VMEM((EW,), jnp.int32),
            pltpu.VMEM((NP,), jnp.float32),
            pltpu.VMEM((SL,), jnp.float32),
            pltpu.VMEM((SL,), jnp.float32),
            pltpu.MemorySpace.VMEM_SHARED((NS, NP), jnp.float32),
        ],
        compiler_params=_SC_PARAMS,
    )
    def deg_kernel(ei_hbm, deg_hbm, didx_v, cnt_v, acc_v, tmp_v, cnts_sh):
        c = lax.axis_index("c")
        s = lax.axis_index("s")
        wid = _wid()
        zero16 = jnp.zeros((L,), jnp.float32)
        ones16 = jnp.full((L,), 1.0, jnp.float32)

        def zbody(i, _):
            cnt_v[pl.ds(i * L, L)] = zero16
            return 0
        lax.fori_loop(0, NP // L, zbody, 0)

        _load_edge_slice(ei_hbm, 1, wid, EWr, EW, didx_v, PAD)

        def body(i, _):
            idx = didx_v[pl.ds(i * L, L)]
            plsc.addupdate_scatter(cnt_v, [idx], ones16)
            return 0
        lax.fori_loop(0, EW // L, body, 0)

        pltpu.sync_copy(cnt_v, cnts_sh.at[s])
        plsc.subcore_barrier()

        pltpu.sync_copy(cnts_sh.at[0, pl.ds(s * SL, SL)], acc_v)
        for t in range(1, NS):
            pltpu.sync_copy(cnts_sh.at[t, pl.ds(s * SL, SL)], tmp_v)

            def abody(i, _):
                sl = pl.ds(i * L, L)
                acc_v[sl] = acc_v[sl] + tmp_v[sl]
                return 0
            lax.fori_loop(0, SL // L, abody, 0)
        pltpu.sync_copy(acc_v, deg_hbm.at[c, pl.ds(s * SL, SL)])

    return deg_kernel


def _make_agg_kernel(NP, EWr, EW, H, PAD):
    RPT = NP // NS
    G = EW // C
    assert G % 2 == 0

    @functools.partial(
        pl.kernel, mesh=_MESH,
        out_type=jax.ShapeDtypeStruct((NC, NP, H), jnp.float32),
        scratch_types=[
            pltpu.VMEM((EW,), jnp.int32),
            pltpu.VMEM((EW,), jnp.int32),
            pltpu.VMEM((C,), jnp.int32),
            pltpu.VMEM((C, H), jnp.float32),
            pltpu.VMEM((C, H), jnp.float32),
            pltpu.VMEM((L, H), jnp.float32),
            pltpu.MemorySpace.VMEM_SHARED((NP, H), jnp.float32),
            pltpu.MemorySpace.VMEM_SHARED((NP, H), jnp.float32),
            pltpu.SemaphoreType.DMA,
            pltpu.SemaphoreType.DMA,
        ],
        compiler_params=_SC_PARAMS,
    )
    def agg_kernel(hs_hbm, ei_hbm, out_hbm,
                   sidx_v, didx_v, dc_v, rows0_v, rows1_v, zb_v,
                   hs_sh, acc_sh, sem0, sem1):
        c = lax.axis_index("c")
        s = lax.axis_index("s")
        wid = _wid()
        zero16 = jnp.zeros((L,), jnp.float32)
        for i in range(L):
            for j in range(H // L):
                zb_v[i, pl.ds(j * L, L)] = zero16

        _load_edge_slice(ei_hbm, 0, wid, EWr, EW, sidx_v, PAD)
        _load_edge_slice(ei_hbm, 1, wid, EWr, EW, didx_v, PAD)
        pltpu.sync_copy(hs_hbm.at[pl.ds(s * RPT, RPT)],
                        hs_sh.at[pl.ds(s * RPT, RPT)])

        def zc(i, _):
            pltpu.sync_copy(zb_v, acc_sh.at[pl.ds(s * RPT + i * L, L)])
            return 0
        lax.fori_loop(0, RPT // L, zc, 0)
        plsc.subcore_barrier()

        def scatter(g, rows_v):
            for j in range(C // L):
                dc_v[pl.ds(j * L, L)] = didx_v[pl.ds(g * C + j * L, L)]
            pltpu.sync_copy(rows_v, acc_sh.at[dc_v], add=True)

        pltpu.async_copy(hs_sh.at[sidx_v.at[pl.ds(0, C)]], rows0_v, sem0)

        def body(i, _):
            g0 = 2 * i
            g1 = g0 + 1
            g2 = g0 + 2
            pltpu.async_copy(hs_sh.at[sidx_v.at[pl.ds(g1 * C, C)]],
                             rows1_v, sem1)
            pltpu.make_async_copy(hs_sh.at[sidx_v.at[pl.ds(g0 * C, C)]],
                                  rows0_v, sem0).wait()
            scatter(g0, rows0_v)

            @pl.when(g2 < G)
            def _():
                pltpu.async_copy(hs_sh.at[sidx_v.at[pl.ds(g2 * C, C)]],
                                 rows0_v, sem0)
            pltpu.make_async_copy(hs_sh.at[sidx_v.at[pl.ds(g1 * C, C)]],
                                  rows1_v, sem1).wait()
            scatter(g1, rows1_v)
            return 0
        lax.fori_loop(0, G // 2, body, 0)
        plsc.subcore_barrier()

        pltpu.sync_copy(acc_sh.at[pl.ds(s * RPT, RPT)],
                        out_hbm.at[c, pl.ds(s * RPT, RPT)])

    return agg_kernel


def _make_scores_kernel(NP, EWr, EW, H, PAD):
    G = EW // C
    assert G % 2 == 0
    NSUB = C // L

    @functools.partial(
        pl.kernel, mesh=_MESH,
        out_type=jax.ShapeDtypeStruct((NW * EW,), jnp.float32),
        scratch_types=[
            pltpu.VMEM((EW,), jnp.int32),
            pltpu.VMEM((EW,), jnp.int32),
            pltpu.VMEM((C, H), jnp.float32),
            pltpu.VMEM((C, H), jnp.float32),
            pltpu.VMEM((C, H), jnp.float32),
            pltpu.VMEM((C, H), jnp.float32),
            pltpu.VMEM((C,), jnp.float32),
            pltpu.MemorySpace.VMEM_SHARED((NP, H), jnp.float32),
            pltpu.SemaphoreType.DMA,
            pltpu.SemaphoreType.DMA,
        ],
        compiler_params=_SC_PARAMS,
    )
    def scores_kernel(nr_hbm, ei_hbm, out_hbm,
                      aidx_v, bidx_v, ab0_v, bb0_v, ab1_v, bb1_v, sc_v,
                      nr_sh, sem0, sem1):
        s = lax.axis_index("s")
        wid = _wid()
        base = wid * EW
        lanes = lax.iota(jnp.int32, L)
        RPT = NP // NS

        def start(g, abuf, bbuf, sem):
            pltpu.async_copy(nr_sh.at[aidx_v.at[pl.ds(g * C, C)]], abuf, sem)
            pltpu.async_copy(nr_sh.at[bidx_v.at[pl.ds(g * C, C)]], bbuf, sem)

        def wait(g, abuf, bbuf, sem):
            pltpu.make_async_copy(nr_sh.at[aidx_v.at[pl.ds(g * C, C)]],
                                  abuf, sem).wait()
            pltpu.make_async_copy(nr_sh.at[bidx_v.at[pl.ds(g * C, C)]],
                                  bbuf, sem).wait()

        def compute(g, abuf, bbuf):
            def sub_body(sub, _):
                acc = jnp.zeros((L,), jnp.float32)
                for j in range(L):
                    e = sub * L + j
                    p = jnp.zeros((L,), jnp.float32)
                    for q in range(H // L):
                        sl = pl.ds(q * L, L)
                        p = p + abuf[e, sl] * bbuf[e, sl]
                    t = jnp.sum(p)
                    acc = jnp.where(lanes == j, t, acc)
                sc_v[pl.ds(sub * L, L)] = acc
                return 0
            lax.fori_loop(0, NSUB, sub_body, 0)
            pltpu.sync_copy(sc_v, out_hbm.at[pl.ds(base + g * C, C)])

        _load_edge_slice(ei_hbm, 0, wid, EWr, EW, aidx_v, PAD)
        _load_edge_slice(ei_hbm, 1, wid, EWr, EW, bidx_v, PAD)
        pltpu.sync_copy(nr_hbm.at[pl.ds(s * RPT, RPT)],
                        nr_sh.at[pl.ds(s * RPT, RPT)])
        plsc.subcore_barrier()
        start(0, ab0_v, bb0_v, sem0)

        def body(i, _):
            g0 = 2 * i
            g1 = g0 + 1
            g2 = g0 + 2
            start(g1, ab1_v, bb1_v, sem1)
            wait(g0, ab0_v, bb0_v, sem0)
            compute(g0, ab0_v, bb0_v)

            @pl.when(g2 < G)
            def _():
                start(g2, ab0_v, bb0_v, sem0)
            wait(g1, ab1_v, bb1_v, sem1)
            compute(g1, ab1_v, bb1_v)
            return 0
        lax.fori_loop(0, G // 2, body, 0)

    return scores_kernel


def _dense1_body(deg2_ref, xp_ref, w1_ref, hs1_ref):
    deg = deg2_ref[0] + deg2_ref[1] + 1.0
    dinv = lax.rsqrt(deg)
    h = jnp.dot(xp_ref[...], w1_ref[...], preferred_element_type=jnp.float32)
    hs1_ref[...] = h * dinv


def _dense2_body(deg2_ref, acc_ref, hs1_ref, w2_ref, b1_ref, hs2_ref):
    dinv = lax.rsqrt(deg2_ref[0] + deg2_ref[1] + 1.0)
    tot = acc_ref[0] + acc_ref[1] + hs1_ref[...]
    out1 = jnp.maximum(dinv * tot + b1_ref[...], 0.0)
    h2 = jnp.dot(out1, w2_ref[...], preferred_element_type=jnp.float32)
    hs2_ref[...] = h2 * dinv


def _dense3_body(deg2_ref, acc_ref, hs2_ref, b2_ref, wd1_ref, bd1_ref,
                 wd2_ref, bd2_ref, nr_ref, recon_ref):
    dinv = lax.rsqrt(deg2_ref[0] + deg2_ref[1] + 1.0)
    tot = acc_ref[0] + acc_ref[1] + hs2_ref[...]
    nr = jnp.maximum(dinv * tot + b2_ref[...], 0.0)
    nr_ref[...] = nr
    r = jnp.maximum(
        jnp.dot(nr, wd1_ref[...], preferred_element_type=jnp.float32)
        + bd1_ref[...], 0.0)
    recon_ref[...] = (
        jnp.dot(r, wd2_ref[...], preferred_element_type=jnp.float32)
        + bd2_ref[...])


def kernel(x, edge_index, batch, W1, b1, W2, b2, Wd1, bd1, Wd2, bd2):
    N, F = x.shape
    H = W1.shape[1]
    E = edge_index.shape[1]

    NP = ((N + NW * L - 1) // (NW * L)) * (NW * L)
    assert E % NW == 0
    EWr = E // NW
    EW = ((EWr + 2 * C - 1) // (2 * C)) * 2 * C

    ei = edge_index.astype(jnp.int32)
    xp = jnp.pad(x, ((0, NP - N), (0, 0)))

    deg_k = _make_deg_kernel(NP, EWr, EW, N)
    agg_k = _make_agg_kernel(NP, EWr, EW, H, N)
    scores_k = _make_scores_kernel(NP, EWr, EW, H, N)

    deg2 = deg_k(ei)
    deg2c = deg2.reshape(NC, NP, 1)

    hs1 = pl.pallas_call(
        _dense1_body,
        out_shape=jax.ShapeDtypeStruct((NP, H), jnp.float32))(
        deg2c, xp, W1)

    acc1 = agg_k(hs1, ei)

    hs2 = pl.pallas_call(
        _dense2_body,
        out_shape=jax.ShapeDtypeStruct((NP, H), jnp.float32))(
        deg2c, acc1, hs1, W2, b1.reshape(1, H))

    acc2 = agg_k(hs2, ei)

    nr, recon = pl.pallas_call(
        _dense3_body,
        out_shape=(jax.ShapeDtypeStruct((NP, H), jnp.float32),
                   jax.ShapeDtypeStruct((NP, F), jnp.float32)))(
        deg2c, acc2, hs2, b2.reshape(1, H),
        Wd1, bd1.reshape(1, H), Wd2, bd2.reshape(1, F))

    scores = scores_k(nr, ei)
    scores = scores.reshape(NW, EW)[:, :EWr].reshape(E)

    return (recon[:N], scores, nr[:N])
